# Initial kernel scaffold; baseline (speedup 1.0000x reference)
import jax
import jax.numpy as jnp
from jax.experimental import pallas as pl


def kernel(feature_vals, level_locs, intercepts):
    raise NotImplementedError("write your pallas kernel here")


# speedup vs baseline: 1.7282x; 1.7282x over previous
import functools

import jax
import jax.numpy as jnp
from jax import lax
from jax.experimental import pallas as pl
from jax.experimental.pallas import tpu as pltpu
from jax.experimental.pallas import tpu_sc as plsc

B = 16384
C = 10
L = 100000
NW = 32
BPW = B // NW
ELEMS = BPW * C
CHUNK = 128
NCH = ELEMS // CHUNK
PAT = 208


def _body(fv_hbm, table_hbm, pat_hbm, out_hbm, fv_v, idx_v, dest_v, pat_v, sem):
    wid = lax.axis_index("s") * 2 + lax.axis_index("c")
    base = pl.multiple_of(wid * BPW, BPW)
    pltpu.sync_copy(fv_hbm.at[pl.ds(base, BPW)], fv_v)
    pltpu.sync_copy(pat_hbm, pat_v)

    lanes = lax.iota(jnp.int32, 16)

    @pl.loop(0, BPW // 16)
    def _build(i):
        off = pl.multiple_of(i * 16, 16)
        fv = fv_v[pl.ds(off, 16)]
        fv = lax.min(lax.max(fv, 0), L - 1)
        bpos10 = (lanes + off) * C
        for c in range(C):
            gidx = fv + c * L
            plsc.store_scatter(idx_v, [bpos10 + c], gidx)

    copies = [
        pltpu.async_copy(
            table_hbm.at[idx_v.at[pl.ds(j * CHUNK, CHUNK)]],
            dest_v.at[pl.ds(j * CHUNK, CHUNK)],
            sem,
        )
        for j in range(NCH)
    ]
    for cp in copies:
        cp.wait()

    @pl.loop(0, NCH)
    def _addint(j):
        jb = pl.multiple_of(j * CHUNK, CHUNK)
        o = pl.multiple_of(lax.rem(j * CHUNK, 80), 16)
        for k in range(CHUNK // 16):
            d = dest_v[pl.ds(jb + k * 16, 16)]
            pv = pat_v[pl.ds(o + k * 16, 16)]
            dest_v[pl.ds(jb + k * 16, 16)] = d + pv

    out_base = pl.multiple_of(wid * ELEMS, ELEMS)
    pltpu.sync_copy(dest_v, out_hbm.at[pl.ds(out_base, ELEMS)])


@jax.jit
def _encode(feature_vals, table_flat, pat):
    run = pl.kernel(
        _body,
        out_type=jax.ShapeDtypeStruct((B * C,), jnp.float32),
        mesh=plsc.VectorSubcoreMesh(core_axis_name="c", subcore_axis_name="s"),
        compiler_params=pltpu.CompilerParams(needs_layout_passes=False),
        scratch_types=[
            pltpu.VMEM((BPW,), jnp.int32),
            pltpu.VMEM((ELEMS,), jnp.int32),
            pltpu.VMEM((ELEMS,), jnp.float32),
            pltpu.VMEM((PAT,), jnp.float32),
            pltpu.SemaphoreType.DMA,
        ],
    )
    return run(feature_vals, table_flat, pat)


def kernel(feature_vals, level_locs, intercepts):
    fv = feature_vals.astype(jnp.int32)
    table_flat = level_locs.reshape(-1)
    pat = jnp.tile(intercepts, (PAT + C - 1) // C)[:PAT]
    out_flat = _encode(fv, table_flat, pat)
    return out_flat.reshape(B, C)


# speedup vs baseline: 1.7730x; 1.0259x over previous
import functools

import jax
import jax.numpy as jnp
from jax import lax
from jax.experimental import pallas as pl
from jax.experimental.pallas import tpu as pltpu
from jax.experimental.pallas import tpu_sc as plsc

B = 16384
C = 10
L = 100000
NW = 32
BPW = B // NW
ELEMS = BPW * C
CHUNK = 128
NCH = ELEMS // CHUNK
PAT = 208


def _body(fv_hbm, table_hbm, int_hbm, out_hbm, fv_v, idx_v, dest_v, int_v, pat_v, sem):
    wid = lax.axis_index("s") * 2 + lax.axis_index("c")
    base = pl.multiple_of(wid * BPW, BPW)
    pltpu.sync_copy(fv_hbm.at[pl.ds(base, BPW)], fv_v)
    pltpu.sync_copy(int_hbm, int_v)

    lanes = lax.iota(jnp.int32, 16)

    for t in range(PAT // 16):
        pv = plsc.load_gather(int_v, [lax.rem(lanes + t * 16, C)])
        pat_v[pl.ds(t * 16, 16)] = pv

    @pl.loop(0, BPW // 16)
    def _build(i):
        off = pl.multiple_of(i * 16, 16)
        fv = fv_v[pl.ds(off, 16)]
        fv = lax.min(lax.max(fv, 0), L - 1)
        bpos10 = (lanes + off) * C
        for c in range(C):
            gidx = fv + c * L
            plsc.store_scatter(idx_v, [bpos10 + c], gidx)

    copies = [
        pltpu.async_copy(
            table_hbm.at[idx_v.at[pl.ds(j * CHUNK, CHUNK)]],
            dest_v.at[pl.ds(j * CHUNK, CHUNK)],
            sem,
        )
        for j in range(NCH)
    ]
    for cp in copies:
        cp.wait()

    @pl.loop(0, NCH)
    def _addint(j):
        jb = pl.multiple_of(j * CHUNK, CHUNK)
        o = pl.multiple_of(lax.rem(j * CHUNK, 80), 16)
        for k in range(CHUNK // 16):
            d = dest_v[pl.ds(jb + k * 16, 16)]
            pv = pat_v[pl.ds(o + k * 16, 16)]
            dest_v[pl.ds(jb + k * 16, 16)] = d + pv

    out_base = pl.multiple_of(wid * ELEMS, ELEMS)
    pltpu.sync_copy(dest_v, out_hbm.at[pl.ds(out_base, ELEMS)])


@jax.jit
def _encode(feature_vals, table_flat, intercepts):
    run = pl.kernel(
        _body,
        out_type=jax.ShapeDtypeStruct((B * C,), jnp.float32),
        mesh=plsc.VectorSubcoreMesh(core_axis_name="c", subcore_axis_name="s"),
        compiler_params=pltpu.CompilerParams(
            needs_layout_passes=False, skip_device_barrier=True
        ),
        scratch_types=[
            pltpu.VMEM((BPW,), jnp.int32),
            pltpu.VMEM((ELEMS,), jnp.int32),
            pltpu.VMEM((ELEMS,), jnp.float32),
            pltpu.VMEM((C,), jnp.float32),
            pltpu.VMEM((PAT,), jnp.float32),
            pltpu.SemaphoreType.DMA,
        ],
    )
    return run(feature_vals, table_flat, intercepts)


def kernel(feature_vals, level_locs, intercepts):
    fv = feature_vals.astype(jnp.int32)
    table_flat = level_locs.reshape(-1)
    out_flat = _encode(fv, table_flat, intercepts)
    return out_flat.reshape(B, C)


# speedup vs baseline: 2.0877x; 1.1775x over previous
import functools

import jax
import jax.numpy as jnp
from jax import lax
from jax.experimental import pallas as pl
from jax.experimental.pallas import tpu as pltpu
from jax.experimental.pallas import tpu_sc as plsc

B = 16384
C = 10
L = 100000
NW = 32
BPW = B // NW


def _body(fv_hbm, table_hbm, int_hbm, out_hbm, fv_v, idx_v, cm_v, out_v, int_v, sem):
    wid = lax.axis_index("s") * 2 + lax.axis_index("c")
    base = pl.multiple_of(wid * BPW, BPW)
    pltpu.sync_copy(fv_hbm.at[pl.ds(base, BPW)], fv_v)
    pltpu.sync_copy(int_hbm, int_v.at[pl.ds(0, C)])

    lanes = lax.iota(jnp.int32, 16)

    @pl.loop(0, BPW // 16)
    def _build(i):
        off = pl.multiple_of(i * 16, 16)
        fv = fv_v[pl.ds(off, 16)]
        fv = lax.min(lax.max(fv, 0), L - 1)
        for c in range(C):
            idx_v[pl.ds(c * BPW + off, 16)] = fv + c * L

    copies = [
        pltpu.async_copy(
            table_hbm.at[idx_v.at[pl.ds(j * 128, 128)]],
            cm_v.at[pl.ds(j * 128, 128)],
            sem,
        )
        for j in range(BPW * C // 128)
    ]
    for cp in copies:
        cp.wait()

    iv = int_v[pl.ds(0, 16)]
    for c in range(C):
        ivec = jnp.broadcast_to(iv[c], (16,))
        cvec = lanes * 0 + c

        @pl.loop(0, BPW // 16)
        def _emit(m, c=c, ivec=ivec, cvec=cvec):
            off = pl.multiple_of(c * BPW + m * 16, 16)
            val = cm_v[pl.ds(off, 16)] + ivec
            plsc.store_scatter(out_v, [m * 16 + lanes, cvec], val)

    pltpu.sync_copy(out_v, out_hbm.at[pl.ds(base, BPW)])


@jax.jit
def _encode(feature_vals, table_flat, intercepts):
    run = pl.kernel(
        _body,
        out_type=jax.ShapeDtypeStruct((B, C), jnp.float32),
        mesh=plsc.VectorSubcoreMesh(core_axis_name="c", subcore_axis_name="s"),
        compiler_params=pltpu.CompilerParams(
            needs_layout_passes=False, skip_device_barrier=True
        ),
        scratch_types=[
            pltpu.VMEM((BPW,), jnp.int32),
            pltpu.VMEM((BPW * C,), jnp.int32),
            pltpu.VMEM((BPW * C,), jnp.float32),
            pltpu.VMEM((BPW, C), jnp.float32),
            pltpu.VMEM((16,), jnp.float32),
            pltpu.SemaphoreType.DMA,
        ],
    )
    return run(feature_vals, table_flat, intercepts)


def kernel(feature_vals, level_locs, intercepts):
    return _encode(
        feature_vals.astype(jnp.int32), level_locs.reshape(-1), intercepts
    )


# speedup vs baseline: 2.7454x; 1.3151x over previous
import functools

import jax
import jax.numpy as jnp
from jax import lax
from jax.experimental import pallas as pl
from jax.experimental.pallas import tpu as pltpu
from jax.experimental.pallas import tpu_sc as plsc

B = 16384
C = 10
L = 100000
NW = 32
BPW = B // NW
CHUNK = 128


def _body(fv_hbm, table_hbm, int_hbm, out_hbm, fv_v, cm_v, int_v, sem):
    wid = lax.axis_index("s") * 2 + lax.axis_index("c")
    base = pl.multiple_of(wid * BPW, BPW)
    pltpu.sync_copy(fv_hbm.at[pl.ds(base, BPW)], fv_v)
    pltpu.sync_copy(int_hbm, int_v.at[pl.ds(0, C)])

    @pl.loop(0, BPW // 16)
    def _clamp(i):
        off = pl.multiple_of(i * 16, 16)
        fv = fv_v[pl.ds(off, 16)]
        fv_v[pl.ds(off, 16)] = lax.min(lax.max(fv, 0), L - 1)

    copies = [
        pltpu.async_copy(
            table_hbm.at[pl.ds(c * L, L)].at[fv_v.at[pl.ds(k * CHUNK, CHUNK)]],
            cm_v.at[c, pl.ds(k * CHUNK, CHUNK)],
            sem,
        )
        for c in range(C)
        for k in range(BPW // CHUNK)
    ]
    for cp in copies:
        cp.wait()

    iv = int_v[pl.ds(0, 16)]
    for c in range(C):
        ivec = jnp.broadcast_to(iv[c], (16,))

        @pl.loop(0, BPW // 16)
        def _add(m, c=c, ivec=ivec):
            off = pl.multiple_of(m * 16, 16)
            cm_v[c, pl.ds(off, 16)] = cm_v[c, pl.ds(off, 16)] + ivec

    pltpu.sync_copy(cm_v, out_hbm.at[pl.ds(0, C), pl.ds(base, BPW)])


@jax.jit
def _encode(feature_vals, table_flat, intercepts):
    run = pl.kernel(
        _body,
        out_type=jax.ShapeDtypeStruct((C, B), jnp.float32),
        mesh=plsc.VectorSubcoreMesh(core_axis_name="c", subcore_axis_name="s"),
        compiler_params=pltpu.CompilerParams(
            needs_layout_passes=False, skip_device_barrier=True
        ),
        scratch_types=[
            pltpu.VMEM((BPW,), jnp.int32),
            pltpu.VMEM((C, BPW), jnp.float32),
            pltpu.VMEM((16,), jnp.float32),
            pltpu.SemaphoreType.DMA,
        ],
    )
    return run(feature_vals, table_flat, intercepts)


def kernel(feature_vals, level_locs, intercepts):
    out_cm = _encode(
        feature_vals.astype(jnp.int32), level_locs.reshape(-1), intercepts
    )
    return out_cm.T


# speedup vs baseline: 2.9363x; 1.0695x over previous
import functools

import jax
import jax.numpy as jnp
from jax import lax
from jax.experimental import pallas as pl
from jax.experimental.pallas import tpu as pltpu
from jax.experimental.pallas import tpu_sc as plsc

B = 16384
C = 10
L = 100000
NW = 32
BPW = B // NW
CHUNK = 128
NCH = BPW // CHUNK


def _body(fv_hbm, table_hbm, int_hbm, out_hbm, fv_v, cm_v, int_v, gsem, osem):
    wid = lax.axis_index("s") * 2 + lax.axis_index("c")
    base = pl.multiple_of(wid * BPW, BPW)
    fv_cp = pltpu.async_copy(fv_hbm.at[pl.ds(base, BPW)], fv_v, osem)
    pltpu.sync_copy(int_hbm, int_v.at[pl.ds(0, C)])
    fv_cp.wait()

    @pl.loop(0, BPW // 16, unroll=4)
    def _clamp(i):
        off = pl.multiple_of(i * 16, 16)
        fv = fv_v[pl.ds(off, 16)]
        fv_v[pl.ds(off, 16)] = lax.min(lax.max(fv, 0), L - 1)

    copies = [
        [
            pltpu.async_copy(
                table_hbm.at[pl.ds(c * L, L)].at[
                    fv_v.at[pl.ds(k * CHUNK, CHUNK)]
                ],
                cm_v.at[c, pl.ds(k * CHUNK, CHUNK)],
                gsem.at[c],
            )
            for k in range(NCH)
        ]
        for c in range(C)
    ]

    iv = int_v[pl.ds(0, 16)]
    out_cps = []
    for c in range(C):
        for cp in copies[c]:
            cp.wait()
        ivec = jnp.broadcast_to(iv[c], (16,))

        @pl.loop(0, BPW // 16, unroll=4)
        def _add(m, c=c, ivec=ivec):
            off = pl.multiple_of(m * 16, 16)
            cm_v[c, pl.ds(off, 16)] = cm_v[c, pl.ds(off, 16)] + ivec

        out_cps.append(
            pltpu.async_copy(
                cm_v.at[c], out_hbm.at[c, pl.ds(base, BPW)], osem
            )
        )
    for cp in out_cps:
        cp.wait()


@jax.jit
def _encode(feature_vals, table_flat, intercepts):
    run = pl.kernel(
        _body,
        out_type=jax.ShapeDtypeStruct((C, B), jnp.float32),
        mesh=plsc.VectorSubcoreMesh(core_axis_name="c", subcore_axis_name="s"),
        compiler_params=pltpu.CompilerParams(
            needs_layout_passes=False, skip_device_barrier=True
        ),
        scratch_types=[
            pltpu.VMEM((BPW,), jnp.int32),
            pltpu.VMEM((C, BPW), jnp.float32),
            pltpu.VMEM((16,), jnp.float32),
            pltpu.SemaphoreType.DMA((C,)),
            pltpu.SemaphoreType.DMA,
        ],
    )
    return run(feature_vals, table_flat, intercepts)


def kernel(feature_vals, level_locs, intercepts):
    out_cm = _encode(
        feature_vals.astype(jnp.int32), level_locs.reshape(-1), intercepts
    )
    return out_cm.T
